# baseline (speedup 1.0000x reference)
import jax
import jax.numpy as jnp
from jax.experimental import pallas as pl
from jax.experimental.pallas import tpu as pltpu


def _mlp_kernel(x_ref, w1_ref, b1_ref, w2_ref, b2_ref, w3_ref, b3_ref, o_ref):
    h = jnp.dot(x_ref[...], w1_ref[...], preferred_element_type=jnp.float32)
    h = jnp.tanh((h + b1_ref[...]).astype(jnp.bfloat16))
    h = jnp.dot(h, w2_ref[...], preferred_element_type=jnp.float32)
    h = jnp.tanh((h + b2_ref[...]).astype(jnp.bfloat16))
    o = jnp.dot(h, w3_ref[...], preferred_element_type=jnp.float32)
    o_ref[...] = o + b3_ref[...]


def kernel(inputs, task_indices, W1, b1, W2, b2, W3, b3):
    del task_indices
    n, d_in = inputs.shape
    hidden = W1.shape[1]
    d_out = W3.shape[1]
    bm = 512

    b1r = b1.reshape(1, hidden)
    b2r = b2.reshape(1, hidden)
    b3r = b3.reshape(1, d_out)

    return pl.pallas_call(
        _mlp_kernel,
        grid=(n // bm,),
        in_specs=[
            pl.BlockSpec((bm, d_in), lambda i: (i, 0)),
            pl.BlockSpec((d_in, hidden), lambda i: (0, 0)),
            pl.BlockSpec((1, hidden), lambda i: (0, 0)),
            pl.BlockSpec((hidden, hidden), lambda i: (0, 0)),
            pl.BlockSpec((1, hidden), lambda i: (0, 0)),
            pl.BlockSpec((hidden, d_out), lambda i: (0, 0)),
            pl.BlockSpec((1, d_out), lambda i: (0, 0)),
        ],
        out_specs=pl.BlockSpec((bm, d_out), lambda i: (i, 0)),
        out_shape=jax.ShapeDtypeStruct((n, d_out), jnp.float32),
        compiler_params=pltpu.CompilerParams(
            dimension_semantics=("arbitrary",),
        ),
    )(inputs, W1, b1r, W2, b2r, W3, b3r)


# speedup vs baseline: 1.0096x; 1.0096x over previous
import jax
import jax.numpy as jnp
from jax.experimental import pallas as pl
from jax.experimental.pallas import tpu as pltpu


def _mlp_kernel(x_ref, w1_ref, b1_ref, w2_ref, b2_ref, w3_ref, b3_ref, o_ref):
    h = jnp.dot(x_ref[...], w1_ref[...], preferred_element_type=jnp.float32)
    h = jnp.tanh((h + b1_ref[...]).astype(jnp.bfloat16))
    h = jnp.dot(h, w2_ref[...], preferred_element_type=jnp.float32)
    h = jnp.tanh((h + b2_ref[...]).astype(jnp.bfloat16))
    o = jnp.dot(h, w3_ref[...], preferred_element_type=jnp.float32)
    o_ref[...] = o + b3_ref[...]


def kernel(inputs, task_indices, W1, b1, W2, b2, W3, b3):
    del task_indices
    n, d_in = inputs.shape
    hidden = W1.shape[1]
    d_out = W3.shape[1]
    bm = 1024

    b1r = b1.reshape(1, hidden)
    b2r = b2.reshape(1, hidden)
    b3r = b3.reshape(1, d_out)

    return pl.pallas_call(
        _mlp_kernel,
        grid=(n // bm,),
        in_specs=[
            pl.BlockSpec((bm, d_in), lambda i: (i, 0)),
            pl.BlockSpec((d_in, hidden), lambda i: (0, 0)),
            pl.BlockSpec((1, hidden), lambda i: (0, 0)),
            pl.BlockSpec((hidden, hidden), lambda i: (0, 0)),
            pl.BlockSpec((1, hidden), lambda i: (0, 0)),
            pl.BlockSpec((hidden, d_out), lambda i: (0, 0)),
            pl.BlockSpec((1, d_out), lambda i: (0, 0)),
        ],
        out_specs=pl.BlockSpec((bm, d_out), lambda i: (i, 0)),
        out_shape=jax.ShapeDtypeStruct((n, d_out), jnp.float32),
        compiler_params=pltpu.CompilerParams(
            dimension_semantics=("arbitrary",),
        ),
    )(inputs, W1, b1r, W2, b2r, W3, b3r)


# speedup vs baseline: 1.0158x; 1.0061x over previous
import jax
import jax.numpy as jnp
from jax.experimental import pallas as pl
from jax.experimental.pallas import tpu as pltpu


def _mlp_kernel(x_ref, w1_hbm, b1_ref, w2_hbm, b2_ref, w3_hbm, b3_ref, o_ref,
                w1s, w2s, w3s, sem1, sem2, sem3):
    c1 = pltpu.make_async_copy(w1_hbm, w1s, sem1)
    c2 = pltpu.make_async_copy(w2_hbm, w2s, sem2)
    c3 = pltpu.make_async_copy(w3_hbm, w3s, sem3)

    @pl.when(pl.program_id(0) == 0)
    def _start_weight_copies():
        c1.start()
        c2.start()
        c3.start()

    @pl.when(pl.program_id(0) == 0)
    def _wait_w1():
        c1.wait()

    h = jnp.dot(x_ref[...], w1s[...], preferred_element_type=jnp.float32)
    h = jnp.tanh((h + b1_ref[...]).astype(jnp.bfloat16))

    @pl.when(pl.program_id(0) == 0)
    def _wait_w2():
        c2.wait()

    h = jnp.dot(h, w2s[...], preferred_element_type=jnp.float32)
    h = jnp.tanh((h + b2_ref[...]).astype(jnp.bfloat16))

    @pl.when(pl.program_id(0) == 0)
    def _wait_w3():
        c3.wait()

    o = jnp.dot(h, w3s[...], preferred_element_type=jnp.float32)
    o_ref[...] = o + b3_ref[...]


def kernel(inputs, task_indices, W1, b1, W2, b2, W3, b3):
    del task_indices
    n, d_in = inputs.shape
    hidden = W1.shape[1]
    d_out = W3.shape[1]
    bm = 1024

    b1r = b1.reshape(1, hidden)
    b2r = b2.reshape(1, hidden)
    b3r = b3.reshape(1, d_out)

    return pl.pallas_call(
        _mlp_kernel,
        grid=(n // bm,),
        in_specs=[
            pl.BlockSpec((bm, d_in), lambda i: (i, 0)),
            pl.BlockSpec(memory_space=pltpu.MemorySpace.HBM),
            pl.BlockSpec((1, hidden), lambda i: (0, 0)),
            pl.BlockSpec(memory_space=pltpu.MemorySpace.HBM),
            pl.BlockSpec((1, hidden), lambda i: (0, 0)),
            pl.BlockSpec(memory_space=pltpu.MemorySpace.HBM),
            pl.BlockSpec((1, d_out), lambda i: (0, 0)),
        ],
        out_specs=pl.BlockSpec((bm, d_out), lambda i: (i, 0)),
        out_shape=jax.ShapeDtypeStruct((n, d_out), jnp.float32),
        scratch_shapes=[
            pltpu.VMEM((d_in, hidden), jnp.float32),
            pltpu.VMEM((hidden, hidden), jnp.float32),
            pltpu.VMEM((hidden, d_out), jnp.float32),
            pltpu.SemaphoreType.DMA,
            pltpu.SemaphoreType.DMA,
            pltpu.SemaphoreType.DMA,
        ],
        compiler_params=pltpu.CompilerParams(
            dimension_semantics=("arbitrary",),
        ),
    )(inputs, W1, b1r, W2, b2r, W3, b3r)


# speedup vs baseline: 1.0158x; 1.0000x over previous
import jax
import jax.numpy as jnp
from jax.experimental import pallas as pl
from jax.experimental.pallas import tpu as pltpu


def _mlp_kernel(x_ref, w1_hbm, b1_ref, w2_hbm, b2_ref, w3_hbm, b3_ref, o_ref,
                w1s, w2s, w3s, sem1, sem2, sem3):
    c1 = pltpu.make_async_copy(w1_hbm, w1s, sem1)
    c2 = pltpu.make_async_copy(w2_hbm, w2s, sem2)
    c3 = pltpu.make_async_copy(w3_hbm, w3s, sem3)

    @pl.when(pl.program_id(0) == 0)
    def _start_weight_copies():
        c1.start()
        c2.start()
        c3.start()

    @pl.when(pl.program_id(0) == 0)
    def _wait_w1():
        c1.wait()

    h = jnp.dot(x_ref[...], w1s[...], preferred_element_type=jnp.float32)
    h = jnp.tanh((h + b1_ref[...]).astype(jnp.bfloat16))

    @pl.when(pl.program_id(0) == 0)
    def _wait_w2():
        c2.wait()

    h = jnp.dot(h, w2s[...], preferred_element_type=jnp.float32)
    h = jnp.tanh((h + b2_ref[...]).astype(jnp.bfloat16))

    @pl.when(pl.program_id(0) == 0)
    def _wait_w3():
        c3.wait()

    o = jnp.dot(h, w3s[...], preferred_element_type=jnp.float32)
    o_ref[...] = o + b3_ref[...]


def kernel(inputs, task_indices, W1, b1, W2, b2, W3, b3):
    del task_indices
    n, d_in = inputs.shape
    hidden = W1.shape[1]
    d_out = W3.shape[1]
    bm = 1024

    b1r = b1.reshape(1, hidden)
    b2r = b2.reshape(1, hidden)
    b3r = b3.reshape(1, d_out)

    return pl.pallas_call(
        _mlp_kernel,
        grid=(n // bm,),
        in_specs=[
            pl.BlockSpec((bm, d_in), lambda i: (i, 0)),
            pl.BlockSpec(memory_space=pltpu.MemorySpace.HBM),
            pl.BlockSpec((1, hidden), lambda i: (0, 0)),
            pl.BlockSpec(memory_space=pltpu.MemorySpace.HBM),
            pl.BlockSpec((1, hidden), lambda i: (0, 0)),
            pl.BlockSpec(memory_space=pltpu.MemorySpace.HBM),
            pl.BlockSpec((1, d_out), lambda i: (0, 0)),
        ],
        out_specs=pl.BlockSpec((bm, d_out), lambda i: (i, 0)),
        out_shape=jax.ShapeDtypeStruct((n, d_out), jnp.float32),
        scratch_shapes=[
            pltpu.VMEM((d_in, hidden), jnp.float32),
            pltpu.VMEM((hidden, hidden), jnp.float32),
            pltpu.VMEM((hidden, d_out), jnp.float32),
            pltpu.SemaphoreType.DMA,
            pltpu.SemaphoreType.DMA,
            pltpu.SemaphoreType.DMA,
        ],
        compiler_params=pltpu.CompilerParams(
            dimension_semantics=("parallel",),
        ),
    )(inputs, W1, b1r, W2, b2r, W3, b3r)
